# Initial kernel scaffold; baseline (speedup 1.0000x reference)
import jax
import jax.numpy as jnp
from jax import lax
from jax.experimental import pallas as pl
from jax.experimental.pallas import tpu as pltpu
from jax.experimental.pallas import tpu_sc as plsc

N_NODES = 10000
N_EDGES = 160000
D = 256
H = 128
NC = 2
NS = 16
CHUNK = 128
NCHUNKS = N_EDGES // CHUNK
ROWS_PER_SUB = N_NODES // NS
BM = 1000
GRID_M = N_NODES // BM


def _sc_mesh():
    return plsc.VectorSubcoreMesh(core_axis_name="c", subcore_axis_name="s")


def _deg_body(edges, ones_hbm, zrows, out, ones_v, idx_v, acc):
    c = lax.axis_index("c")
    s = lax.axis_index("s")
    w = c * NS + s
    pltpu.sync_copy(ones_hbm, ones_v)
    pltpu.sync_copy(zrows, acc.at[pl.ds(s * ROWS_PER_SUB, ROWS_PER_SUB), :])
    plsc.subcore_barrier()
    iters = (NCHUNKS + NC * NS - 1) // (NC * NS)

    def body(i, carry):
        k = w + i * (NC * NS)

        @pl.when(k < NCHUNKS)
        def _():
            pltpu.sync_copy(edges.at[1, pl.ds(k * CHUNK, CHUNK)], idx_v)
            pltpu.sync_copy(ones_v, acc.at[idx_v], add=True)

        return carry

    lax.fori_loop(0, iters, body, 0)
    plsc.subcore_barrier()
    pltpu.sync_copy(acc.at[pl.ds(s * ROWS_PER_SUB, ROWS_PER_SUB), :],
                    out.at[c, pl.ds(s * ROWS_PER_SUB, ROWS_PER_SUB), :])


def _sc_deg(edge_index, ones16, z16):
    return pl.kernel(
        _deg_body,
        out_type=jax.ShapeDtypeStruct((NC, N_NODES, 16), jnp.float32),
        mesh=_sc_mesh(),
        scratch_types=[
            pltpu.VMEM((CHUNK, 16), jnp.float32),
            pltpu.VMEM((CHUNK,), jnp.int32),
            pltpu.VMEM_SHARED((N_NODES, 16), jnp.float32),
        ],
    )(edge_index, ones16, z16)


def _scatter_body(g2d, edges, zrows, out, idx_s, idx_d, rows_v, acc, sem):
    c = lax.axis_index("c")
    s = lax.axis_index("s")
    pltpu.sync_copy(zrows, acc.at[pl.ds(s * ROWS_PER_SUB, ROWS_PER_SUB), :])
    plsc.subcore_barrier()
    base = c * N_NODES
    iters = (NCHUNKS + NS - 1) // NS

    def body(i, carry):
        k = s + i * NS

        @pl.when(k < NCHUNKS)
        def _():
            pltpu.sync_copy(edges.at[0, pl.ds(k * CHUNK, CHUNK)], idx_s)
            pltpu.sync_copy(edges.at[1, pl.ds(k * CHUNK, CHUNK)], idx_d)
            for j in range(CHUNK // 16):
                sl = pl.ds(j * 16, 16)
                idx_s[sl] = idx_s[sl] + base
            pltpu.async_copy(g2d.at[idx_s], rows_v, sem).wait()
            pltpu.sync_copy(rows_v, acc.at[idx_d], add=True)

        return carry

    lax.fori_loop(0, iters, body, 0)
    plsc.subcore_barrier()
    pltpu.sync_copy(acc.at[pl.ds(s * ROWS_PER_SUB, ROWS_PER_SUB), :],
                    out.at[c, pl.ds(s * ROWS_PER_SUB, ROWS_PER_SUB), :])


def _sc_scatter(g2d, edge_index, z128):
    return pl.kernel(
        _scatter_body,
        out_type=jax.ShapeDtypeStruct((NC, N_NODES, H), jnp.float32),
        mesh=_sc_mesh(),
        scratch_types=[
            pltpu.VMEM((CHUNK,), jnp.int32),
            pltpu.VMEM((CHUNK,), jnp.int32),
            pltpu.VMEM((CHUNK, H), jnp.float32),
            pltpu.VMEM_SHARED((N_NODES, H), jnp.float32),
            pltpu.SemaphoreType.DMA,
        ],
    )(g2d, edge_index, z128)


def _dinv(dcnt_ref):
    deg = dcnt_ref[0][:, :1] + dcnt_ref[1][:, :1] + 1.0
    return lax.rsqrt(deg)


def _gelu(x):
    return 0.5 * x * (1.0 + lax.erf(x * 0.7071067811865476))


def _first_body(x_ref, w_ref, dcnt_ref, out_ref):
    dinv = _dinv(dcnt_ref)
    h = jnp.dot(x_ref[...], w_ref[...], preferred_element_type=jnp.float32)
    g = dinv * h
    out_ref[0] = g[:, :H]
    out_ref[1] = g[:, H:]


def _mid_body(acc_ref, g_ref, dcnt_ref, b_ref, w_ref, out_ref):
    dinv = _dinv(dcnt_ref)
    pre = jnp.concatenate([acc_ref[0] + g_ref[0], acc_ref[1] + g_ref[1]], axis=1)
    a = _gelu(dinv * pre + b_ref[...])
    hn = jnp.dot(a, w_ref[...], preferred_element_type=jnp.float32)
    gn = dinv * hn
    out_ref[0] = gn[:, :H]
    out_ref[1] = gn[:, H:]


def _final_body(acc_ref, g_ref, dcnt_ref, b_ref, out_ref):
    dinv = _dinv(dcnt_ref)
    pre = jnp.concatenate([acc_ref[0] + g_ref[0], acc_ref[1] + g_ref[1]], axis=1)
    out_ref[...] = dinv * pre + b_ref[...]


_DCNT_SPEC = pl.BlockSpec((NC, BM, 16), lambda i: (0, i, 0))
_HALVES_SPEC = pl.BlockSpec((NC, BM, H), lambda i: (0, i, 0))
_W_SPEC = pl.BlockSpec((D, D), lambda i: (0, 0))
_B_SPEC = pl.BlockSpec((1, D), lambda i: (0, 0))


def _tc_first(x, W1, dcnt):
    return pl.pallas_call(
        _first_body,
        grid=(GRID_M,),
        in_specs=[pl.BlockSpec((BM, D), lambda i: (i, 0)), _W_SPEC, _DCNT_SPEC],
        out_specs=_HALVES_SPEC,
        out_shape=jax.ShapeDtypeStruct((NC, N_NODES, H), jnp.float32),
    )(x, W1, dcnt)


def _tc_mid(acc, g, dcnt, b, W):
    return pl.pallas_call(
        _mid_body,
        grid=(GRID_M,),
        in_specs=[_HALVES_SPEC, _HALVES_SPEC, _DCNT_SPEC, _B_SPEC, _W_SPEC],
        out_specs=_HALVES_SPEC,
        out_shape=jax.ShapeDtypeStruct((NC, N_NODES, H), jnp.float32),
    )(acc, g, dcnt, b, W)


def _tc_final(acc, g, dcnt, b):
    return pl.pallas_call(
        _final_body,
        grid=(GRID_M,),
        in_specs=[_HALVES_SPEC, _HALVES_SPEC, _DCNT_SPEC, _B_SPEC],
        out_specs=pl.BlockSpec((BM, D), lambda i: (i, 0)),
        out_shape=jax.ShapeDtypeStruct((N_NODES, D), jnp.float32),
    )(acc, g, dcnt, b)


def kernel(x, edge_index, W1, b1, W2, b2, W3, b3):
    ones16 = jnp.ones((CHUNK, 16), jnp.float32)
    z16 = jnp.zeros((ROWS_PER_SUB, 16), jnp.float32)
    z128 = jnp.zeros((ROWS_PER_SUB, H), jnp.float32)
    dcnt = _sc_deg(edge_index, ones16, z16)
    g1 = _tc_first(x, W1, dcnt)
    acc1 = _sc_scatter(g1.reshape(NC * N_NODES, H), edge_index, z128)
    g2 = _tc_mid(acc1, g1, dcnt, b1.reshape(1, D), W2)
    acc2 = _sc_scatter(g2.reshape(NC * N_NODES, H), edge_index, z128)
    g3 = _tc_mid(acc2, g2, dcnt, b2.reshape(1, D), W3)
    acc3 = _sc_scatter(g3.reshape(NC * N_NODES, H), edge_index, z128)
    return _tc_final(acc3, g3, dcnt, b3.reshape(1, D))


# speedup vs baseline: 8.7598x; 8.7598x over previous
import jax
import jax.numpy as jnp
from jax import lax
from jax.experimental import pallas as pl
from jax.experimental.pallas import tpu as pltpu
from jax.experimental.pallas import tpu_sc as plsc

N_NODES = 10000
N_EDGES = 160000
D = 256
H = 128
NC = 2
NS = 16
CHUNK = 128
NCHUNKS = N_EDGES // CHUNK
N_PAD = 10240
ROWS_PER_SUB = N_PAD // NS
BM = 1000
GRID_M = N_NODES // BM


def _sc_mesh():
    return plsc.VectorSubcoreMesh(core_axis_name="c", subcore_axis_name="s")


def _deg_body(edges, ones_hbm, zrows, out, ones_v, idx_v, acc):
    c = lax.axis_index("c")
    s = lax.axis_index("s")
    w = c * NS + s
    pltpu.sync_copy(ones_hbm, ones_v)
    pltpu.sync_copy(zrows, acc.at[pl.ds(s * ROWS_PER_SUB, ROWS_PER_SUB), :])
    plsc.subcore_barrier()
    iters = (NCHUNKS + NC * NS - 1) // (NC * NS)

    def body(i, carry):
        k = w + i * (NC * NS)

        @pl.when(k < NCHUNKS)
        def _():
            pltpu.sync_copy(edges.at[1, pl.ds(k * CHUNK, CHUNK)], idx_v)
            pltpu.sync_copy(ones_v, acc.at[idx_v], add=True)

        return carry

    lax.fori_loop(0, iters, body, 0)
    plsc.subcore_barrier()
    pltpu.sync_copy(acc.at[pl.ds(s * ROWS_PER_SUB, ROWS_PER_SUB), :],
                    out.at[c, pl.ds(s * ROWS_PER_SUB, ROWS_PER_SUB), :])


def _sc_deg(edge_index, ones128, z128):
    return pl.kernel(
        _deg_body,
        out_type=jax.ShapeDtypeStruct((NC, N_PAD, H), jnp.float32),
        mesh=_sc_mesh(),
        scratch_types=[
            pltpu.VMEM((CHUNK, H), jnp.float32),
            pltpu.VMEM((CHUNK,), jnp.int32),
            pltpu.VMEM_SHARED((N_PAD, H), jnp.float32),
        ],
    )(edge_index, ones128, z128)


def _scatter_body(g2d, edges, zrows, out, idx_s, idx_d, rows_v, acc, sem):
    c = lax.axis_index("c")
    s = lax.axis_index("s")
    pltpu.sync_copy(zrows, acc.at[pl.ds(s * ROWS_PER_SUB, ROWS_PER_SUB), :])
    plsc.subcore_barrier()
    base = c * N_PAD
    iters = (NCHUNKS + NS - 1) // NS

    def body(i, carry):
        k = s + i * NS

        @pl.when(k < NCHUNKS)
        def _():
            pltpu.sync_copy(edges.at[0, pl.ds(k * CHUNK, CHUNK)], idx_s)
            pltpu.sync_copy(edges.at[1, pl.ds(k * CHUNK, CHUNK)], idx_d)
            for j in range(CHUNK // 16):
                sl = pl.ds(j * 16, 16)
                idx_s[sl] = idx_s[sl] + base
            pltpu.async_copy(g2d.at[idx_s], rows_v, sem).wait()
            pltpu.sync_copy(rows_v, acc.at[idx_d], add=True)

        return carry

    lax.fori_loop(0, iters, body, 0)
    plsc.subcore_barrier()
    pltpu.sync_copy(acc.at[pl.ds(s * ROWS_PER_SUB, ROWS_PER_SUB), :],
                    out.at[c, pl.ds(s * ROWS_PER_SUB, ROWS_PER_SUB), :])


def _sc_scatter(g2d, edge_index, z128):
    return pl.kernel(
        _scatter_body,
        out_type=jax.ShapeDtypeStruct((NC, N_PAD, H), jnp.float32),
        mesh=_sc_mesh(),
        scratch_types=[
            pltpu.VMEM((CHUNK,), jnp.int32),
            pltpu.VMEM((CHUNK,), jnp.int32),
            pltpu.VMEM((CHUNK, H), jnp.float32),
            pltpu.VMEM_SHARED((N_PAD, H), jnp.float32),
            pltpu.SemaphoreType.DMA,
        ],
    )(g2d, edge_index, z128)


def _dinv(dcnt_ref):
    deg = dcnt_ref[0][:, :1] + dcnt_ref[1][:, :1] + 1.0
    return lax.rsqrt(deg)


def _gelu(x):
    return 0.5 * x * (1.0 + lax.erf(x * 0.7071067811865476))


def _first_body(x_ref, w_ref, dcnt_ref, out_ref):
    dinv = _dinv(dcnt_ref)
    h = jnp.dot(x_ref[...], w_ref[...], preferred_element_type=jnp.float32)
    g = dinv * h
    out_ref[0] = g[:, :H]
    out_ref[1] = g[:, H:]


def _mid_body(acc_ref, g_ref, dcnt_ref, b_ref, w_ref, out_ref):
    dinv = _dinv(dcnt_ref)
    pre = jnp.concatenate([acc_ref[0] + g_ref[0], acc_ref[1] + g_ref[1]], axis=1)
    a = _gelu(dinv * pre + b_ref[...])
    hn = jnp.dot(a, w_ref[...], preferred_element_type=jnp.float32)
    gn = dinv * hn
    out_ref[0] = gn[:, :H]
    out_ref[1] = gn[:, H:]


def _final_body(acc_ref, g_ref, dcnt_ref, b_ref, out_ref):
    dinv = _dinv(dcnt_ref)
    pre = jnp.concatenate([acc_ref[0] + g_ref[0], acc_ref[1] + g_ref[1]], axis=1)
    out_ref[...] = dinv * pre + b_ref[...]


_DCNT_SPEC = pl.BlockSpec((NC, BM, H), lambda i: (0, i, 0))
_HALVES_SPEC = pl.BlockSpec((NC, BM, H), lambda i: (0, i, 0))
_W_SPEC = pl.BlockSpec((D, D), lambda i: (0, 0))
_B_SPEC = pl.BlockSpec((1, D), lambda i: (0, 0))


def _tc_first(x, W1, dcnt):
    return pl.pallas_call(
        _first_body,
        grid=(GRID_M,),
        in_specs=[pl.BlockSpec((BM, D), lambda i: (i, 0)), _W_SPEC, _DCNT_SPEC],
        out_specs=_HALVES_SPEC,
        out_shape=jax.ShapeDtypeStruct((NC, N_PAD, H), jnp.float32),
    )(x, W1, dcnt)


def _tc_mid(acc, g, dcnt, b, W):
    return pl.pallas_call(
        _mid_body,
        grid=(GRID_M,),
        in_specs=[_HALVES_SPEC, _HALVES_SPEC, _DCNT_SPEC, _B_SPEC, _W_SPEC],
        out_specs=_HALVES_SPEC,
        out_shape=jax.ShapeDtypeStruct((NC, N_PAD, H), jnp.float32),
    )(acc, g, dcnt, b, W)


def _tc_final(acc, g, dcnt, b):
    return pl.pallas_call(
        _final_body,
        grid=(GRID_M,),
        in_specs=[_HALVES_SPEC, _HALVES_SPEC, _DCNT_SPEC, _B_SPEC],
        out_specs=pl.BlockSpec((BM, D), lambda i: (i, 0)),
        out_shape=jax.ShapeDtypeStruct((N_NODES, D), jnp.float32),
    )(acc, g, dcnt, b)


def kernel(x, edge_index, W1, b1, W2, b2, W3, b3):
    ones128 = jnp.ones((CHUNK, H), jnp.float32)
    z128 = jnp.zeros((ROWS_PER_SUB, H), jnp.float32)
    dcnt = _sc_deg(edge_index, ones128, z128)
    g1 = _tc_first(x, W1, dcnt)
    acc1 = _sc_scatter(g1.reshape(NC * N_PAD, H), edge_index, z128)
    g2 = _tc_mid(acc1, g1, dcnt, b1.reshape(1, D), W2)
    acc2 = _sc_scatter(g2.reshape(NC * N_PAD, H), edge_index, z128)
    g3 = _tc_mid(acc2, g2, dcnt, b2.reshape(1, D), W3)
    acc3 = _sc_scatter(g3.reshape(NC * N_PAD, H), edge_index, z128)
    return _tc_final(acc3, g3, dcnt, b3.reshape(1, D))
